# Initial kernel scaffold; baseline (speedup 1.0000x reference)
import jax
import jax.numpy as jnp
from jax.experimental import pallas as pl


def kernel(x, neighbor_indices, distancesq, W1, b1, W2, b2):
    raise NotImplementedError("write your pallas kernel here")


# speedup vs baseline: 5.2817x; 5.2817x over previous
import functools

import jax
import jax.numpy as jnp
from jax import lax
from jax.experimental import pallas as pl
from jax.experimental.pallas import tpu as pltpu
from jax.experimental.pallas import tpu_sc as plsc

_K = 16
_F = 64
_L = 16
_NC = 2
_NS = 16
_NW = _NC * _NS
_C = 32
_CK = _C * _K


def _mm_relu(x, w, b, block_rows=512):
    vp, d = x.shape
    f = w.shape[1]
    assert vp % block_rows == 0

    def body(x_ref, w_ref, b_ref, o_ref):
        acc = jnp.dot(x_ref[...], w_ref[...], preferred_element_type=jnp.float32)
        o_ref[...] = jnp.maximum(acc + b_ref[...], 0.0)

    return pl.pallas_call(
        body,
        grid=(vp // block_rows,),
        in_specs=[
            pl.BlockSpec((block_rows, d), lambda i: (i, 0)),
            pl.BlockSpec((d, f), lambda i: (0, 0)),
            pl.BlockSpec((1, f), lambda i: (0, 0)),
        ],
        out_specs=pl.BlockSpec((block_rows, f), lambda i: (i, 0)),
        out_shape=jax.ShapeDtypeStruct((vp, f), jnp.float32),
    )(x, w, b.reshape(1, f))


@functools.lru_cache(maxsize=None)
def _make_knn(vp: int):
    n_w = vp // _NW
    n_chunks = n_w // _C
    assert n_chunks * _C == n_w
    mesh = plsc.VectorSubcoreMesh(core_axis_name="c", subcore_axis_name="s")

    @functools.partial(
        pl.kernel,
        out_type=jax.ShapeDtypeStruct((vp, 2 * _F), jnp.float32),
        mesh=mesh,
        compiler_params=pltpu.CompilerParams(use_tc_tiling_on_sc=False),
        scratch_types=[
            pltpu.VMEM((_CK,), jnp.int32),
            pltpu.VMEM((_CK,), jnp.float32),
            pltpu.VMEM((_CK, _F), jnp.float32),
            pltpu.VMEM((_C, _F), jnp.float32),
            pltpu.VMEM((_C, 2 * _F), jnp.float32),
            pltpu.SemaphoreType.DMA,
        ],
    )
    def knn(feat_hbm, idx_hbm, dsq_hbm, out_hbm,
            idx_v, dsq_v, rows_v, self_v, out_v, sem):
        wid = lax.axis_index("s") * _NC + lax.axis_index("c")
        w_base = wid * n_w

        def chunk_body(ci, carry):
            base = w_base + ci * _C
            pltpu.sync_copy(idx_hbm.at[pl.ds(base * _K, _CK)], idx_v)
            pltpu.sync_copy(dsq_hbm.at[pl.ds(base * _K, _CK)], dsq_v)
            pltpu.sync_copy(feat_hbm.at[pl.ds(base, _C)], self_v)
            pltpu.async_copy(feat_hbm.at[idx_v], rows_v, sem).wait()

            def vert_body(v, c2):
                wv = jnp.exp(dsq_v[pl.ds(v * _K, _K)] * -10.0)
                r0 = v * _K
                w0 = wv[0]
                acc_m = []
                acc_x = []
                for fb in range(_F // _L):
                    g = rows_v[r0, pl.ds(fb * _L, _L)] * w0
                    acc_m.append(g)
                    acc_x.append(g)
                for k in range(1, _K):
                    wk = wv[k]
                    for fb in range(_F // _L):
                        g = rows_v[r0 + k, pl.ds(fb * _L, _L)] * wk
                        acc_m[fb] = acc_m[fb] + g
                        acc_x[fb] = jnp.maximum(acc_x[fb], g)
                for fb in range(_F // _L):
                    s = self_v[v, pl.ds(fb * _L, _L)]
                    out_v[v, pl.ds(fb * _L, _L)] = acc_m[fb] * (1.0 / _K) - s
                    out_v[v, pl.ds(_F + fb * _L, _L)] = acc_x[fb] - s
                return c2

            lax.fori_loop(0, _C, vert_body, 0)
            pltpu.sync_copy(out_v, out_hbm.at[pl.ds(base, _C)])
            return carry

        lax.fori_loop(0, n_chunks, chunk_body, 0)

    return knn


def kernel(x, neighbor_indices, distancesq, W1, b1, W2, b2):
    v, d = x.shape
    chunk = _NW * _C
    vp = ((v + chunk - 1) // chunk) * chunk
    pad = vp - v
    xp = jnp.pad(x, ((0, pad), (0, 0)))
    idx_flat = jnp.pad(neighbor_indices, ((0, pad), (0, 0))).reshape(-1)
    dsq_flat = jnp.pad(distancesq, ((0, pad), (0, 0))).reshape(-1)

    knn = _make_knn(vp)
    h1 = _mm_relu(xp, W1, b1)
    out1 = knn(h1, idx_flat, dsq_flat)
    h2 = _mm_relu(out1, W2, b2)
    out2 = knn(h2, idx_flat, dsq_flat)
    return jnp.concatenate([out1[:v], out2[:v], x], axis=-1)
